# baseline (speedup 1.0000x reference)
import functools
import jax
import jax.numpy as jnp
from jax import lax
from jax.experimental import pallas as pl
from jax.experimental.pallas import tpu as pltpu
from jax.experimental.pallas import tpu_sc as plsc

_VOCAB = 1000000
_DIM = 64
_B = 16384
_K = 20
_NC = 2
_NS = 16
_NW = _NC * _NS
_BPW = _B // _NW
_CB = 16
_NCHUNK = _BPW // _CB
_NROWS = _CB * _K


def _sc_scores_body(v_hbm, u_hbm, ctx_hbm, tgt_hbm, neg_hbm,
                    pos_hbm, negs_hbm,
                    idx_c, idx_t, idx_n, v_rows, u_rows, n_buf,
                    pos_buf, neg_buf, sem_vu, sem_n):
  wid = lax.axis_index("s") * _NC + lax.axis_index("c")
  base = wid * _BPW

  pltpu.sync_copy(ctx_hbm.at[pl.ds(base, _BPW)], idx_c)
  pltpu.sync_copy(tgt_hbm.at[pl.ds(base, _BPW)], idx_t)
  pltpu.sync_copy(neg_hbm.at[pl.ds(base * _K, _BPW * _K)], idx_n)

  for off in range(0, _BPW, 128):
    pltpu.async_copy(v_hbm.at[idx_c.at[pl.ds(off, 128)]],
                     v_rows.at[pl.ds(off, 128)], sem_vu)
    pltpu.async_copy(u_hbm.at[idx_t.at[pl.ds(off, 128)]],
                     u_rows.at[pl.ds(off, 128)], sem_vu)
  pltpu.make_async_copy(v_hbm.at[pl.ds(0, _BPW)], v_rows, sem_vu).wait()
  pltpu.make_async_copy(u_hbm.at[pl.ds(0, _BPW)], u_rows, sem_vu).wait()

  lanes = lax.iota(jnp.int32, 16)
  rows_k = [lanes * _K + k for k in range(_K)]

  def chunk_body(ci, carry):
    ib = ci * _NROWS
    pltpu.async_copy(u_hbm.at[idx_n.at[pl.ds(ib, 128)]],
                     n_buf.at[pl.ds(0, 128)], sem_n)
    pltpu.async_copy(u_hbm.at[idx_n.at[pl.ds(ib + 128, 128)]],
                     n_buf.at[pl.ds(128, 128)], sem_n)
    pltpu.async_copy(u_hbm.at[idx_n.at[pl.ds(ib + 256, 64)]],
                     n_buf.at[pl.ds(256, 64)], sem_n)
    pltpu.make_async_copy(u_hbm.at[pl.ds(0, _NROWS)], n_buf, sem_n).wait()

    row_vu = lanes + ci * _CB

    def d_body(d, dcarry):
      acc_p, acc_n = dcarry
      d_vec = jnp.full((16,), d, dtype=jnp.int32)
      v_col = plsc.load_gather(v_rows, [row_vu, d_vec])
      u_col = plsc.load_gather(u_rows, [row_vu, d_vec])
      acc_p = acc_p + u_col * v_col
      for k in range(_K):
        n_col = plsc.load_gather(n_buf, [rows_k[k], d_vec])
        acc_n = acc_n + n_col * v_col
      return acc_p, acc_n

    zero = jnp.zeros((16,), jnp.float32)
    acc_p, acc_n = lax.fori_loop(0, _DIM, d_body, (zero, zero))
    plsc.store_scatter(pos_buf, [row_vu], acc_p)
    plsc.store_scatter(neg_buf, [row_vu], -acc_n)
    return carry

  lax.fori_loop(0, _NCHUNK, chunk_body, 0)

  pltpu.sync_copy(pos_buf, pos_hbm.at[pl.ds(base, _BPW)])
  pltpu.sync_copy(neg_buf, negs_hbm.at[pl.ds(base, _BPW)])


_sc_scores = functools.partial(
    pl.kernel,
    out_type=(jax.ShapeDtypeStruct((_B,), jnp.float32),
              jax.ShapeDtypeStruct((_B,), jnp.float32)),
    mesh=plsc.VectorSubcoreMesh(core_axis_name="c", subcore_axis_name="s",
                                num_cores=_NC, num_subcores=_NS),
    scratch_types=[
        pltpu.VMEM((_BPW,), jnp.int32),
        pltpu.VMEM((_BPW,), jnp.int32),
        pltpu.VMEM((_BPW * _K,), jnp.int32),
        pltpu.VMEM((_BPW, _DIM), jnp.float32),
        pltpu.VMEM((_BPW, _DIM), jnp.float32),
        pltpu.VMEM((_NROWS, _DIM), jnp.float32),
        pltpu.VMEM((_BPW,), jnp.float32),
        pltpu.VMEM((_BPW,), jnp.float32),
        pltpu.SemaphoreType.DMA,
        pltpu.SemaphoreType.DMA,
    ],
    compiler_params=pltpu.CompilerParams(needs_layout_passes=False,
                                         use_tc_tiling_on_sc=False),
)(_sc_scores_body)


def _loss_body(p_ref, n_ref, o_ref):
  def logsig(z):
    return jnp.minimum(z, 0.0) - jnp.log1p(jnp.exp(-jnp.abs(z)))
  total = jnp.sum(logsig(p_ref[...]) + logsig(n_ref[...]))
  o_ref[...] = jnp.full((1, 1), -total * (1.0 / _B), dtype=jnp.float32)


def kernel(ctx_words, target_words, neg_words, V, U):
  ctx = ctx_words.reshape(-1).astype(jnp.int32)
  tgt = target_words.reshape(-1).astype(jnp.int32)
  neg = neg_words.reshape(-1).astype(jnp.int32)
  pos_s, neg_s = _sc_scores(V, U, ctx, tgt, neg)
  out = pl.pallas_call(
      _loss_body,
      out_shape=jax.ShapeDtypeStruct((1, 1), jnp.float32),
  )(pos_s.reshape(128, 128), neg_s.reshape(128, 128))
  return out[0, 0]


# speedup vs baseline: 2.6405x; 2.6405x over previous
import functools
import jax
import jax.numpy as jnp
from jax import lax
from jax.experimental import pallas as pl
from jax.experimental.pallas import tpu as pltpu
from jax.experimental.pallas import tpu_sc as plsc

_VOCAB = 1000000
_DIM = 64
_B = 16384
_K = 20
_NC = 2
_NS = 16
_NW = _NC * _NS
_BPW = _B // _NW
_CB = 16
_NCHUNK = _BPW // _CB
_NROWS = _CB * _K
_NI = _BPW * _K


def _sc_scores_body(v_hbm, u_hbm, ctx_hbm, tgt_hbm, neg_hbm,
                    pos_hbm, negs_hbm,
                    idx_c, idx_t, idx_n, v_rows, u_rows, n_buf,
                    tpb, tnb, pos_buf, neg_buf, sem_a, sem_b):
  wid = lax.axis_index("s") * _NC + lax.axis_index("c")
  base = wid * _BPW

  pltpu.sync_copy(ctx_hbm.at[pl.ds(base, _BPW)], idx_c)
  pltpu.sync_copy(tgt_hbm.at[pl.ds(base, _BPW)], idx_t)
  pltpu.sync_copy(neg_hbm.at[pl.ds(base * _K, _NI)], idx_n)

  lanes = lax.iota(jnp.int32, 16)

  def fire(ci, sem):
    slot = ci & 1
    ib = ci * _NROWS
    nb = slot * _NROWS
    pltpu.async_copy(v_hbm.at[idx_c.at[pl.ds(ci * _CB, _CB)]],
                     v_rows.at[pl.ds(slot * _CB, _CB)], sem)
    pltpu.async_copy(u_hbm.at[idx_t.at[pl.ds(ci * _CB, _CB)]],
                     u_rows.at[pl.ds(slot * _CB, _CB)], sem)
    pltpu.async_copy(u_hbm.at[idx_n.at[pl.ds(ib, 128)]],
                     n_buf.at[pl.ds(nb, 128)], sem)
    pltpu.async_copy(u_hbm.at[idx_n.at[pl.ds(ib + 128, 128)]],
                     n_buf.at[pl.ds(nb + 128, 128)], sem)
    pltpu.async_copy(u_hbm.at[idx_n.at[pl.ds(ib + 256, 64)]],
                     n_buf.at[pl.ds(nb + 256, 64)], sem)

  def drain(sem):
    pltpu.make_async_copy(v_hbm.at[pl.ds(0, _CB)],
                          v_rows.at[pl.ds(0, _CB)], sem).wait()
    pltpu.make_async_copy(u_hbm.at[pl.ds(0, _CB)],
                          u_rows.at[pl.ds(0, _CB)], sem).wait()
    pltpu.make_async_copy(u_hbm.at[pl.ds(0, _NROWS)],
                          n_buf.at[pl.ds(0, _NROWS)], sem).wait()

  fire(0, sem_a)

  def chunk_body(ci, carry):
    slot = ci & 1

    @pl.when((ci < _NCHUNK - 1) & (slot == 0))
    def _():
      fire(ci + 1, sem_b)

    @pl.when((ci < _NCHUNK - 1) & (slot == 1))
    def _():
      fire(ci + 1, sem_a)

    @pl.when(slot == 0)
    def _():
      drain(sem_a)

    @pl.when(slot == 1)
    def _():
      drain(sem_b)

    vu0 = slot * _CB
    nr0 = slot * _NROWS

    def b_body(b, bcarry):
      bv = vu0 + b
      v = [v_rows[bv, pl.ds(j * 16, 16)] for j in range(4)]
      u = [u_rows[bv, pl.ds(j * 16, 16)] for j in range(4)]
      tp = ((u[0] * v[0] + u[1] * v[1]) + (u[2] * v[2] + u[3] * v[3]))
      nrow = nr0 + b * _K
      tn = [jnp.zeros((16,), jnp.float32) for _ in range(4)]
      for k in range(_K):
        for j in range(4):
          tn[j] = tn[j] + n_buf[nrow + k, pl.ds(j * 16, 16)] * v[j]
      tpb[pl.ds(b * 16, 16)] = tp
      tnb[pl.ds(b * 16, 16)] = (tn[0] + tn[1]) + (tn[2] + tn[3])
      return bcarry

    lax.fori_loop(0, _CB, b_body, 0)

    l16 = lanes * 16
    acc_p = plsc.load_gather(tpb, [l16])
    acc_n = plsc.load_gather(tnb, [l16])
    for l in range(1, 16):
      acc_p = acc_p + plsc.load_gather(tpb, [l16 + l])
      acc_n = acc_n + plsc.load_gather(tnb, [l16 + l])
    bpos = lanes + ci * _CB
    plsc.store_scatter(pos_buf, [bpos], acc_p)
    plsc.store_scatter(neg_buf, [bpos], -acc_n)
    return carry

  lax.fori_loop(0, _NCHUNK, chunk_body, 0)

  pltpu.sync_copy(pos_buf, pos_hbm.at[pl.ds(base, _BPW)])
  pltpu.sync_copy(neg_buf, negs_hbm.at[pl.ds(base, _BPW)])


_sc_scores = functools.partial(
    pl.kernel,
    out_type=(jax.ShapeDtypeStruct((_B,), jnp.float32),
              jax.ShapeDtypeStruct((_B,), jnp.float32)),
    mesh=plsc.VectorSubcoreMesh(core_axis_name="c", subcore_axis_name="s",
                                num_cores=_NC, num_subcores=_NS),
    scratch_types=[
        pltpu.VMEM((_BPW,), jnp.int32),
        pltpu.VMEM((_BPW,), jnp.int32),
        pltpu.VMEM((_NI,), jnp.int32),
        pltpu.VMEM((2 * _CB, 128), jnp.float32),
        pltpu.VMEM((2 * _CB, 128), jnp.float32),
        pltpu.VMEM((2 * _NROWS, 128), jnp.float32),
        pltpu.VMEM((256,), jnp.float32),
        pltpu.VMEM((256,), jnp.float32),
        pltpu.VMEM((_BPW,), jnp.float32),
        pltpu.VMEM((_BPW,), jnp.float32),
        pltpu.SemaphoreType.DMA,
        pltpu.SemaphoreType.DMA,
    ],
    compiler_params=pltpu.CompilerParams(needs_layout_passes=False),
)(_sc_scores_body)


def _loss_body(p_ref, n_ref, o_ref):
  def logsig(z):
    return jnp.minimum(z, 0.0) - jnp.log1p(jnp.exp(-jnp.abs(z)))
  total = jnp.sum(logsig(p_ref[...]) + logsig(n_ref[...]))
  o_ref[...] = jnp.full((1, 1), -total * (1.0 / _B), dtype=jnp.float32)


_RC = 32768


def _repack_body(vt_ref, o_ref):
  xt = jnp.swapaxes(vt_ref[...], 0, 1)
  o_ref[...] = jnp.concatenate(
      [xt, jnp.zeros((_RC, _DIM), jnp.float32)], axis=1)


def _repack(table):
  grid = (_VOCAB + _RC - 1) // _RC
  return pl.pallas_call(
      _repack_body,
      grid=(grid,),
      in_specs=[pl.BlockSpec((_DIM, _RC), lambda j: (0, j))],
      out_specs=pl.BlockSpec((_RC, 128), lambda j: (j, 0)),
      out_shape=jax.ShapeDtypeStruct((_VOCAB, 128), jnp.float32),
  )(jnp.swapaxes(table, 0, 1))


def kernel(ctx_words, target_words, neg_words, V, U):
  ctx = ctx_words.reshape(-1).astype(jnp.int32)
  tgt = target_words.reshape(-1).astype(jnp.int32)
  neg = neg_words.reshape(-1).astype(jnp.int32)
  vp = _repack(V)
  up = _repack(U)
  pos_s, neg_s = _sc_scores(vp, up, ctx, tgt, neg)
  out = pl.pallas_call(
      _loss_body,
      out_shape=jax.ShapeDtypeStruct((1, 1), jnp.float32),
  )(pos_s.reshape(128, 128), neg_s.reshape(128, 128))
  return out[0, 0]
